# Initial kernel scaffold; baseline (speedup 1.0000x reference)
import jax
import jax.numpy as jnp
from jax.experimental import pallas as pl


def kernel(input_seq, adjacency, W, bias, prelu_a):
    raise NotImplementedError("write your pallas kernel here")


# speedup vs baseline: 1.0276x; 1.0276x over previous
import functools

import jax
import jax.numpy as jnp
from jax.experimental import pallas as pl
from jax.experimental.pallas import tpu as pltpu

N = 10000
D_IN = 128
D_OUT = 128
BLK_M = 400


def _gcn_kernel(x_ref, w_ref, b_ref, alpha_ref, a_ref, out_ref, mapped_ref):
    i = pl.program_id(0)

    @pl.when(i == 0)
    def _compute_mapped():
        mapped_ref[...] = jax.lax.dot_general(
            x_ref[0],
            w_ref[...],
            (((1,), (1,)), ((), ())),
            preferred_element_type=jnp.float32,
        ).astype(jnp.bfloat16)

    a_blk = a_ref[0].astype(jnp.bfloat16)
    acc = jnp.dot(a_blk, mapped_ref[...], preferred_element_type=jnp.float32)
    out = acc + b_ref[...]
    alpha = alpha_ref[0]
    out_ref[0] = jnp.where(out >= 0, out, alpha * out)


@jax.jit
def kernel(input_seq, adjacency, W, bias, prelu_a):
    grid = (N // BLK_M,)
    out = pl.pallas_call(
        _gcn_kernel,
        grid=grid,
        in_specs=[
            pl.BlockSpec((1, N, D_IN), lambda i: (0, 0, 0)),
            pl.BlockSpec((D_OUT, D_IN), lambda i: (0, 0)),
            pl.BlockSpec((1, D_OUT), lambda i: (0, 0)),
            pl.BlockSpec(memory_space=pltpu.SMEM),
            pl.BlockSpec((1, BLK_M, N), lambda i: (0, i, 0)),
        ],
        out_specs=pl.BlockSpec((1, BLK_M, D_OUT), lambda i: (0, i, 0)),
        out_shape=jax.ShapeDtypeStruct((1, N, D_OUT), jnp.float32),
        scratch_shapes=[pltpu.VMEM((N, D_OUT), jnp.bfloat16)],
    )(
        input_seq,
        W,
        bias.reshape(1, D_OUT),
        prelu_a.reshape(1),
        adjacency,
    )
    return out


# speedup vs baseline: 1.0391x; 1.0112x over previous
import functools

import jax
import jax.numpy as jnp
from jax.experimental import pallas as pl
from jax.experimental.pallas import tpu as pltpu

N = 10000
D_IN = 128
D_OUT = 128
BLK_M = 400


def _gcn_kernel(x_ref, w_ref, b_ref, alpha_ref, a_ref, out_ref, mapped_ref):
    i = pl.program_id(0)

    @pl.when(i == 0)
    def _compute_mapped():
        mapped_ref[...] = jax.lax.dot_general(
            x_ref[0],
            w_ref[...],
            (((1,), (1,)), ((), ())),
            preferred_element_type=jnp.float32,
        )

    acc = jnp.dot(
        a_ref[0],
        mapped_ref[...],
        preferred_element_type=jnp.float32,
        precision=jax.lax.Precision.DEFAULT,
    )
    out = acc + b_ref[...]
    alpha = alpha_ref[0]
    out_ref[0] = jnp.where(out >= 0, out, alpha * out)


@jax.jit
def kernel(input_seq, adjacency, W, bias, prelu_a):
    grid = (N // BLK_M,)
    out = pl.pallas_call(
        _gcn_kernel,
        grid=grid,
        in_specs=[
            pl.BlockSpec((1, N, D_IN), lambda i: (0, 0, 0)),
            pl.BlockSpec((D_OUT, D_IN), lambda i: (0, 0)),
            pl.BlockSpec((1, D_OUT), lambda i: (0, 0)),
            pl.BlockSpec(memory_space=pltpu.SMEM),
            pl.BlockSpec((1, BLK_M, N), lambda i: (0, i, 0)),
        ],
        out_specs=pl.BlockSpec((1, BLK_M, D_OUT), lambda i: (0, i, 0)),
        out_shape=jax.ShapeDtypeStruct((1, N, D_OUT), jnp.float32),
        scratch_shapes=[pltpu.VMEM((N, D_OUT), jnp.float32)],
    )(
        input_seq,
        W,
        bias.reshape(1, D_OUT),
        prelu_a.reshape(1),
        adjacency,
    )
    return out
